# Initial kernel scaffold; baseline (speedup 1.0000x reference)
import jax
import jax.numpy as jnp
from jax.experimental import pallas as pl


def kernel(features, gt_boxes, gt_classes):
    raise NotImplementedError("write your pallas kernel here")


# speedup vs baseline: 2.5532x; 2.5532x over previous
import jax
import jax.numpy as jnp
from jax.experimental import pallas as pl

_NUM_CLASSES = 599
_OUT_HW = 40
_IMG = 1024.0


def _kern(xf_ref, boxes_ref, cls_ref, rh_ref, rw_ref, out_ref):
    X = xf_ref[0]
    bx = boxes_ref[0]
    cls = cls_ref[0]
    RH = rh_ref[...]
    RW = rw_ref[...]

    s = jnp.float32(_OUT_HW / _IMG)
    rb = jnp.round(bx * s)
    x1 = jnp.maximum(rb[:, 0:1], 0.0)
    y1 = jnp.maximum(rb[:, 1:2], 0.0)
    x2 = jnp.minimum(rb[:, 2:3], float(_OUT_HW))
    y2 = jnp.minimum(rb[:, 3:4], float(_OUT_HW))
    n = bx.shape[0]
    g = jax.lax.broadcasted_iota(jnp.int32, (n, _OUT_HW), 1).astype(jnp.float32)
    my = ((g >= y1) & (g < y2)).astype(jnp.float32)
    mx = ((g >= x1) & (g < x2)).astype(jnp.float32)
    cy = jnp.sum(my, axis=1, keepdims=True)
    cx = jnp.sum(mx, axis=1, keepdims=True)
    vf = ((x1 < x2) & (y1 < y2)).astype(jnp.float32)
    scale_n = vf / jnp.maximum(cy * cx, 1.0)

    wy = jnp.dot(my, RH, preferred_element_type=jnp.float32)
    wx = jnp.dot(mx, RW, preferred_element_type=jnp.float32)
    wf = wy * wx * scale_n
    pooled = jax.lax.dot_general(
        wf, X, (((1,), (1,)), ((), ())),
        preferred_element_type=jnp.float32)

    ki = jax.lax.broadcasted_iota(jnp.int32, (_NUM_CLASSES, n), 0)
    oht = (ki == cls).astype(jnp.float32)
    sums = jnp.dot(oht, pooled, preferred_element_type=jnp.float32)
    counts = jnp.dot(oht, vf, preferred_element_type=jnp.float32)
    out_ref[0] = sums / jnp.maximum(counts, 1.0)


def kernel(features, gt_boxes, gt_classes):
    B, C, H, W = features.shape
    N = gt_boxes.shape[1]
    HW = H * W

    R = jax.image.resize(jnp.eye(H, dtype=jnp.float32), (_OUT_HW, H),
                         method='bilinear', antialias=False)
    RH = jnp.repeat(R, W, axis=1)
    RW = jnp.tile(R, (1, W))

    xf = features.reshape(B, C, HW)
    clsr = gt_classes.astype(jnp.int32).reshape(B, 1, N)

    out = pl.pallas_call(
        _kern,
        grid=(B,),
        in_specs=[
            pl.BlockSpec((1, C, HW), lambda b: (b, 0, 0)),
            pl.BlockSpec((1, N, 4), lambda b: (b, 0, 0)),
            pl.BlockSpec((1, 1, N), lambda b: (b, 0, 0)),
            pl.BlockSpec((_OUT_HW, HW), lambda b: (0, 0)),
            pl.BlockSpec((_OUT_HW, HW), lambda b: (0, 0)),
        ],
        out_specs=pl.BlockSpec((1, _NUM_CLASSES, C), lambda b: (b, 0, 0)),
        out_shape=jax.ShapeDtypeStruct((B, _NUM_CLASSES, C), jnp.float32),
    )(xf, gt_boxes, clsr, RH, RW)
    return out
